# Initial kernel scaffold; baseline (speedup 1.0000x reference)
import jax
import jax.numpy as jnp
from jax.experimental import pallas as pl


def kernel(x, control_points, ln_gamma, ln_beta):
    raise NotImplementedError("write your pallas kernel here")


# speedup vs baseline: 1.4843x; 1.4843x over previous
import functools

import jax
import jax.numpy as jnp
from jax.experimental import pallas as pl


def _ln_body(cp_ref, g_ref, b_ref, o_ref, *, batch):
    h = cp_ref[...]
    mean = jnp.mean(h, axis=-1, keepdims=True)
    c = h - mean
    var = jnp.mean(c * c, axis=-1, keepdims=True)
    y = c * jax.lax.rsqrt(var + 1e-5) * g_ref[...] + b_ref[...]
    o_ref[...] = jnp.broadcast_to(y[None], (batch,) + y.shape)


def kernel(x, control_points, ln_gamma, ln_beta):
    batch, seq_len = x.shape
    d_model = control_points.shape[-1]
    cp = control_points[:seq_len]
    block_s = 512
    grid = (seq_len // block_s,)
    return pl.pallas_call(
        functools.partial(_ln_body, batch=batch),
        grid=grid,
        in_specs=[
            pl.BlockSpec((block_s, d_model), lambda i: (i, 0)),
            pl.BlockSpec((d_model,), lambda i: (0,)),
            pl.BlockSpec((d_model,), lambda i: (0,)),
        ],
        out_specs=pl.BlockSpec((batch, block_s, d_model), lambda i: (0, i, 0)),
        out_shape=jax.ShapeDtypeStruct((batch, seq_len, d_model), jnp.float32),
    )(cp, ln_gamma, ln_beta)
